# baseline (speedup 1.0000x reference)
import functools
import jax
import jax.numpy as jnp
from jax import lax
from jax.experimental import pallas as pl
from jax.experimental.pallas import tpu as pltpu
from jax.experimental.pallas import tpu_sc as plsc

N_NODES = 10000
N_EDGES = 320000
D_FEAT = 128
HIDDEN = 16
N_CLASSES = 32
NEG_SLOPE = 0.2

NC = 2
NS = 16
NW = NC * NS
K = 1024
CH0 = 7
CH1 = 13
T_CHUNKS = NS * (CH0 + CH1)
E_PAD = T_CHUNKS * K
N_ACC = -(-(N_NODES + 1) // (NS * 8)) * NS * 8
RPS = N_ACC // NS


def _leaky(a):
  return jnp.where(a >= 0, a, NEG_SLOPE * a)


def _dense1_body(x_ref, w_ref, att_ref, y_ref):
  xp = jnp.dot(x_ref[...], w_ref[...], preferred_element_type=jnp.float32)
  alpha = jnp.sum(xp * att_ref[...], axis=1, keepdims=True)
  y_ref[...] = xp * _leaky(alpha)


def _dense2_body(p_ref, y1_ref, b1_ref, w_ref, att_ref, y_ref):
  h = p_ref[0] + p_ref[1] + y1_ref[...] + b1_ref[...]
  h = jnp.maximum(h, 0.0)
  xp = jnp.dot(h, w_ref[...], preferred_element_type=jnp.float32)
  alpha = jnp.sum(xp * att_ref[...], axis=1, keepdims=True)
  y_ref[...] = xp * _leaky(alpha)


def _final_body(q_ref, y2_ref, b2_ref, o_ref):
  o_ref[...] = q_ref[0] + q_ref[1] + y2_ref[...] + b2_ref[...]


def _tc_call(body, out_shape, *args):
  return pl.pallas_call(
      body, out_shape=jax.ShapeDtypeStruct(out_shape, jnp.float32))(*args)


def _make_sc_scatter(d):
  mesh = plsc.VectorSubcoreMesh(core_axis_name="c", subcore_axis_name="s")

  @functools.partial(
      pl.kernel,
      out_type=jax.ShapeDtypeStruct((NC, N_ACC, d), jnp.float32),
      mesh=mesh,
      compiler_params=pltpu.CompilerParams(use_tc_tiling_on_sc=False),
      scratch_types=[
          pltpu.VMEM((max(CH0, CH1), K), jnp.int32),
          pltpu.VMEM((max(CH0, CH1), K), jnp.int32),
          pltpu.VMEM((K, d), jnp.float32),
          pltpu.VMEM((RPS, d), jnp.float32),
          pltpu.VMEM_SHARED((N_ACC, d), jnp.float32),
          pltpu.SemaphoreType.DMA,
      ],
  )
  def sc_scatter(y_hbm, src_hbm, dst_hbm, zero_hbm, out_hbm,
                 sidx, didx, rows, stage, acc, sem0):
    c = lax.axis_index("c")
    s = lax.axis_index("s")

    pltpu.sync_copy(zero_hbm, stage)
    pltpu.sync_copy(stage, acc.at[pl.ds(s * RPS, RPS)])
    plsc.subcore_barrier()

    def run(base, nch):
      pltpu.sync_copy(src_hbm.at[pl.ds(base, nch)], sidx.at[pl.ds(0, nch)])
      pltpu.sync_copy(dst_hbm.at[pl.ds(base, nch)], didx.at[pl.ds(0, nch)])

      def batch(j, carry):
        pltpu.async_copy(y_hbm.at[sidx.at[j]], rows, sem0).wait()
        pltpu.sync_copy(rows, acc.at[didx.at[j]], add=True)
        return carry

      lax.fori_loop(0, nch, batch, 0)

    @pl.when(c == 0)
    def _():
      run(s * CH0, CH0)

    @pl.when(c == 1)
    def _():
      run(NS * CH0 + s * CH1, CH1)

    plsc.subcore_barrier()

    pltpu.sync_copy(acc.at[pl.ds(s * RPS, RPS)], stage)
    pltpu.sync_copy(stage, out_hbm.at[c, pl.ds(s * RPS, RPS)])

  return sc_scatter


_sc_scatter_h = _make_sc_scatter(HIDDEN)
_sc_scatter_c = _make_sc_scatter(N_CLASSES)


@jax.jit
def kernel(x, edge_index, edge_weight, W1, att_src1, bias1, W2, att_src2,
           bias2):
  del edge_weight
  src = edge_index[0].astype(jnp.int32)
  dst = edge_index[1].astype(jnp.int32)
  pad = jnp.full((E_PAD - N_EDGES,), N_NODES, jnp.int32)
  src_p = jnp.concatenate([src, pad]).reshape(T_CHUNKS, K)
  dst_p = jnp.concatenate([dst, pad]).reshape(T_CHUNKS, K)

  x_p = jnp.concatenate(
      [x, jnp.zeros((N_ACC - N_NODES, D_FEAT), jnp.float32)])

  zeros_h = jnp.zeros((RPS, HIDDEN), jnp.float32)
  zeros_c = jnp.zeros((RPS, N_CLASSES), jnp.float32)

  y1 = _tc_call(_dense1_body, (N_ACC, HIDDEN), x_p, W1,
                att_src1.reshape(1, HIDDEN))
  p = _sc_scatter_h(y1, src_p, dst_p, zeros_h)
  y2 = _tc_call(_dense2_body, (N_ACC, N_CLASSES), p, y1,
                bias1.reshape(1, HIDDEN), W2, att_src2.reshape(1, N_CLASSES))
  q = _sc_scatter_c(y2, src_p, dst_p, zeros_c)
  out = _tc_call(_final_body, (N_ACC, N_CLASSES), q, y2,
                 bias2.reshape(1, N_CLASSES))
  return out[:N_NODES]


# speedup vs baseline: 1.1161x; 1.1161x over previous
import functools
import jax
import jax.numpy as jnp
from jax import lax
from jax.experimental import pallas as pl
from jax.experimental.pallas import tpu as pltpu
from jax.experimental.pallas import tpu_sc as plsc

N_NODES = 10000
N_EDGES = 320000
D_FEAT = 128
HIDDEN = 16
N_CLASSES = 32
NEG_SLOPE = 0.2

NC = 2
NS = 16
NW = NC * NS
K = 1024
CH0 = 13
CH1 = 7
T_CHUNKS = NS * (CH0 + CH1)
E_PAD = T_CHUNKS * K
N_ACC = -(-(N_NODES + 1) // (NS * 8)) * NS * 8
RPS = N_ACC // NS


def _leaky(a):
  return jnp.where(a >= 0, a, NEG_SLOPE * a)


def _dense1_body(x_ref, w_ref, att_ref, y_ref):
  xp = jnp.dot(x_ref[...], w_ref[...], preferred_element_type=jnp.float32)
  alpha = jnp.sum(xp * att_ref[...], axis=1, keepdims=True)
  y_ref[...] = xp * _leaky(alpha)


def _dense2_body(p_ref, y1_ref, b1_ref, w_ref, att_ref, y_ref):
  h = p_ref[0] + p_ref[1] + y1_ref[...] + b1_ref[...]
  h = jnp.maximum(h, 0.0)
  xp = jnp.dot(h, w_ref[...], preferred_element_type=jnp.float32)
  alpha = jnp.sum(xp * att_ref[...], axis=1, keepdims=True)
  y_ref[...] = xp * _leaky(alpha)


def _final_body(q_ref, y2_ref, b2_ref, o_ref):
  o_ref[...] = q_ref[0] + q_ref[1] + y2_ref[...] + b2_ref[...]


def _tc_call(body, out_shape, *args):
  return pl.pallas_call(
      body, out_shape=jax.ShapeDtypeStruct(out_shape, jnp.float32))(*args)


def _make_sc_scatter(d):
  mesh = plsc.VectorSubcoreMesh(core_axis_name="c", subcore_axis_name="s")

  @functools.partial(
      pl.kernel,
      out_type=jax.ShapeDtypeStruct((NC, N_ACC, d), jnp.float32),
      mesh=mesh,
      compiler_params=pltpu.CompilerParams(use_tc_tiling_on_sc=False),
      scratch_types=[
          pltpu.VMEM((max(CH0, CH1), K), jnp.int32),
          pltpu.VMEM((max(CH0, CH1), K), jnp.int32),
          pltpu.VMEM((K, d), jnp.float32),
          pltpu.VMEM((RPS, d), jnp.float32),
          pltpu.VMEM_SHARED((N_ACC, d), jnp.float32),
          pltpu.SemaphoreType.DMA,
      ],
  )
  def sc_scatter(y_hbm, src_hbm, dst_hbm, zero_hbm, out_hbm,
                 sidx, didx, rows, stage, acc, sem0):
    c = lax.axis_index("c")
    s = lax.axis_index("s")

    pltpu.sync_copy(zero_hbm, stage)
    pltpu.sync_copy(stage, acc.at[pl.ds(s * RPS, RPS)])
    plsc.subcore_barrier()

    def run(base, nch):
      pltpu.sync_copy(src_hbm.at[pl.ds(base, nch)], sidx.at[pl.ds(0, nch)])
      pltpu.sync_copy(dst_hbm.at[pl.ds(base, nch)], didx.at[pl.ds(0, nch)])

      def batch(j, carry):
        pltpu.async_copy(y_hbm.at[sidx.at[j]], rows, sem0).wait()
        pltpu.sync_copy(rows, acc.at[didx.at[j]], add=True)
        return carry

      lax.fori_loop(0, nch, batch, 0)

    @pl.when(c == 0)
    def _():
      run(s * CH0, CH0)

    @pl.when(c == 1)
    def _():
      run(NS * CH0 + s * CH1, CH1)

    plsc.subcore_barrier()

    pltpu.sync_copy(acc.at[pl.ds(s * RPS, RPS)], stage)
    pltpu.sync_copy(stage, out_hbm.at[c, pl.ds(s * RPS, RPS)])

  return sc_scatter


_sc_scatter_h = _make_sc_scatter(HIDDEN)
_sc_scatter_c = _make_sc_scatter(N_CLASSES)


@jax.jit
def kernel(x, edge_index, edge_weight, W1, att_src1, bias1, W2, att_src2,
           bias2):
  del edge_weight
  src = edge_index[0].astype(jnp.int32)
  dst = edge_index[1].astype(jnp.int32)
  pad = jnp.full((E_PAD - N_EDGES,), N_NODES, jnp.int32)
  src_p = jnp.concatenate([src, pad]).reshape(T_CHUNKS, K)
  dst_p = jnp.concatenate([dst, pad]).reshape(T_CHUNKS, K)

  x_p = jnp.concatenate(
      [x, jnp.zeros((N_ACC - N_NODES, D_FEAT), jnp.float32)])

  zeros_h = jnp.zeros((RPS, HIDDEN), jnp.float32)
  zeros_c = jnp.zeros((RPS, N_CLASSES), jnp.float32)

  y1 = _tc_call(_dense1_body, (N_ACC, HIDDEN), x_p, W1,
                att_src1.reshape(1, HIDDEN))
  p = _sc_scatter_h(y1, src_p, dst_p, zeros_h)
  y2 = _tc_call(_dense2_body, (N_ACC, N_CLASSES), p, y1,
                bias1.reshape(1, HIDDEN), W2, att_src2.reshape(1, N_CLASSES))
  q = _sc_scatter_c(y2, src_p, dst_p, zeros_c)
  out = _tc_call(_final_body, (N_ACC, N_CLASSES), q, y2,
                 bias2.reshape(1, N_CLASSES))
  return out[:N_NODES]


# speedup vs baseline: 1.6921x; 1.5161x over previous
import functools
import jax
import jax.numpy as jnp
from jax import lax
from jax.experimental import pallas as pl
from jax.experimental.pallas import tpu as pltpu
from jax.experimental.pallas import tpu_sc as plsc

N_NODES = 10000
N_EDGES = 320000
D_FEAT = 128
HIDDEN = 16
N_CLASSES = 32
NEG_SLOPE = 0.2

NC = 2
NS = 16
NW = NC * NS
K = 1024
CH0 = 13
CH1 = 7
T_CHUNKS = NS * (CH0 + CH1)
E_PAD = T_CHUNKS * K
N_ACC = -(-(N_NODES + 1) // (NS * 8)) * NS * 8
RPS = N_ACC // NS


def _leaky(a):
  return jnp.where(a >= 0, a, NEG_SLOPE * a)


def _dense1_body(x_ref, w_ref, att_ref, y_ref):
  xp = jnp.dot(x_ref[...], w_ref[...], preferred_element_type=jnp.float32)
  alpha = jnp.sum(xp * att_ref[...], axis=1, keepdims=True)
  y_ref[...] = xp * _leaky(alpha)


def _dense2_body(p_ref, y1_ref, b1_ref, w_ref, att_ref, y_ref):
  h = p_ref[0] + p_ref[1] + y1_ref[...] + b1_ref[...]
  h = jnp.maximum(h, 0.0)
  xp = jnp.dot(h, w_ref[...], preferred_element_type=jnp.float32)
  alpha = jnp.sum(xp * att_ref[...], axis=1, keepdims=True)
  y_ref[...] = xp * _leaky(alpha)


def _final_body(q_ref, y2_ref, b2_ref, o_ref):
  o_ref[...] = q_ref[0] + q_ref[1] + y2_ref[...] + b2_ref[...]


def _tc_call(body, out_shape, *args):
  return pl.pallas_call(
      body, out_shape=jax.ShapeDtypeStruct(out_shape, jnp.float32))(*args)


def _make_sc_scatter(d):
  mesh = plsc.VectorSubcoreMesh(core_axis_name="c", subcore_axis_name="s")

  @functools.partial(
      pl.kernel,
      out_type=jax.ShapeDtypeStruct((NC, N_ACC, d), jnp.float32),
      mesh=mesh,
      compiler_params=pltpu.CompilerParams(use_tc_tiling_on_sc=False),
      scratch_types=[
          pltpu.VMEM((max(CH0, CH1), K), jnp.int32),
          pltpu.VMEM((max(CH0, CH1), K), jnp.int32),
          pltpu.VMEM((K, d), jnp.float32),
          pltpu.VMEM((RPS, d), jnp.float32),
          pltpu.VMEM_SHARED((N_ACC, d), jnp.float32),
          pltpu.VMEM_SHARED((N_ACC, d), jnp.float32),
          pltpu.SemaphoreType.DMA,
      ],
  )
  def sc_scatter(y_hbm, src_hbm, dst_hbm, zero_hbm, out_hbm,
                 sidx, didx, rows, stage, acc, y_sp, sem0):
    c = lax.axis_index("c")
    s = lax.axis_index("s")

    stripe = pl.ds(s * RPS, RPS)
    pltpu.sync_copy(y_hbm.at[stripe], stage)
    pltpu.sync_copy(stage, y_sp.at[stripe])
    pltpu.sync_copy(zero_hbm, stage)
    pltpu.sync_copy(stage, acc.at[stripe])
    plsc.subcore_barrier()

    def run(base, nch):
      pltpu.sync_copy(src_hbm.at[pl.ds(base, nch)], sidx.at[pl.ds(0, nch)])
      pltpu.sync_copy(dst_hbm.at[pl.ds(base, nch)], didx.at[pl.ds(0, nch)])

      def batch(j, carry):
        pltpu.async_copy(y_sp.at[sidx.at[j]], rows, sem0).wait()
        pltpu.sync_copy(rows, acc.at[didx.at[j]], add=True)
        return carry

      lax.fori_loop(0, nch, batch, 0)

    @pl.when(c == 0)
    def _():
      run(s * CH0, CH0)

    @pl.when(c == 1)
    def _():
      run(NS * CH0 + s * CH1, CH1)

    plsc.subcore_barrier()

    pltpu.sync_copy(acc.at[pl.ds(s * RPS, RPS)], stage)
    pltpu.sync_copy(stage, out_hbm.at[c, pl.ds(s * RPS, RPS)])

  return sc_scatter


_sc_scatter_h = _make_sc_scatter(HIDDEN)
_sc_scatter_c = _make_sc_scatter(N_CLASSES)


@jax.jit
def kernel(x, edge_index, edge_weight, W1, att_src1, bias1, W2, att_src2,
           bias2):
  del edge_weight
  src = edge_index[0].astype(jnp.int32)
  dst = edge_index[1].astype(jnp.int32)
  pad = jnp.full((E_PAD - N_EDGES,), N_NODES, jnp.int32)
  src_p = jnp.concatenate([src, pad]).reshape(T_CHUNKS, K)
  dst_p = jnp.concatenate([dst, pad]).reshape(T_CHUNKS, K)

  x_p = jnp.concatenate(
      [x, jnp.zeros((N_ACC - N_NODES, D_FEAT), jnp.float32)])

  zeros_h = jnp.zeros((RPS, HIDDEN), jnp.float32)
  zeros_c = jnp.zeros((RPS, N_CLASSES), jnp.float32)

  y1 = _tc_call(_dense1_body, (N_ACC, HIDDEN), x_p, W1,
                att_src1.reshape(1, HIDDEN))
  p = _sc_scatter_h(y1, src_p, dst_p, zeros_h)
  y2 = _tc_call(_dense2_body, (N_ACC, N_CLASSES), p, y1,
                bias1.reshape(1, HIDDEN), W2, att_src2.reshape(1, N_CLASSES))
  q = _sc_scatter_c(y2, src_p, dst_p, zeros_c)
  out = _tc_call(_final_body, (N_ACC, N_CLASSES), q, y2,
                 bias2.reshape(1, N_CLASSES))
  return out[:N_NODES]


# speedup vs baseline: 1.7551x; 1.0372x over previous
import functools
import jax
import jax.numpy as jnp
from jax import lax
from jax.experimental import pallas as pl
from jax.experimental.pallas import tpu as pltpu
from jax.experimental.pallas import tpu_sc as plsc

N_NODES = 10000
N_EDGES = 320000
D_FEAT = 128
HIDDEN = 16
N_CLASSES = 32
NEG_SLOPE = 0.2

NC = 2
NS = 16
NW = NC * NS
K = 1024
CH0 = 12
CH1 = 8
T_CHUNKS = NS * (CH0 + CH1)
E_PAD = T_CHUNKS * K
N_ACC = -(-(N_NODES + 1) // (NS * 8)) * NS * 8
RPS = N_ACC // NS


def _leaky(a):
  return jnp.where(a >= 0, a, NEG_SLOPE * a)


def _dense1_body(x_ref, w_ref, att_ref, y_ref):
  xp = jnp.dot(x_ref[...], w_ref[...], preferred_element_type=jnp.float32)
  alpha = jnp.sum(xp * att_ref[...], axis=1, keepdims=True)
  y_ref[...] = xp * _leaky(alpha)


def _dense2_body(p_ref, y1_ref, b1_ref, w_ref, att_ref, y_ref):
  h = p_ref[0] + p_ref[1] + y1_ref[...] + b1_ref[...]
  h = jnp.maximum(h, 0.0)
  xp = jnp.dot(h, w_ref[...], preferred_element_type=jnp.float32)
  alpha = jnp.sum(xp * att_ref[...], axis=1, keepdims=True)
  y_ref[...] = xp * _leaky(alpha)


def _final_body(q_ref, y2_ref, b2_ref, o_ref):
  o_ref[...] = q_ref[0] + q_ref[1] + y2_ref[...] + b2_ref[...]


def _tc_call(body, out_shape, *args):
  return pl.pallas_call(
      body, out_shape=jax.ShapeDtypeStruct(out_shape, jnp.float32))(*args)


def _make_sc_scatter(d):
  mesh = plsc.VectorSubcoreMesh(core_axis_name="c", subcore_axis_name="s")

  @functools.partial(
      pl.kernel,
      out_type=jax.ShapeDtypeStruct((NC, N_ACC, d), jnp.float32),
      mesh=mesh,
      compiler_params=pltpu.CompilerParams(use_tc_tiling_on_sc=False),
      scratch_types=[
          pltpu.VMEM((max(CH0, CH1), K), jnp.int32),
          pltpu.VMEM((max(CH0, CH1), K), jnp.int32),
          pltpu.VMEM((K, d), jnp.float32),
          pltpu.VMEM((RPS, d), jnp.float32),
          pltpu.VMEM_SHARED((N_ACC, d), jnp.float32),
          pltpu.VMEM_SHARED((N_ACC, d), jnp.float32),
          pltpu.SemaphoreType.DMA,
      ],
  )
  def sc_scatter(y_hbm, src_hbm, dst_hbm, zero_hbm, out_hbm,
                 sidx, didx, rows, stage, acc, y_sp, sem0):
    c = lax.axis_index("c")
    s = lax.axis_index("s")

    stripe = pl.ds(s * RPS, RPS)
    pltpu.sync_copy(y_hbm.at[stripe], stage)
    pltpu.sync_copy(stage, y_sp.at[stripe])
    pltpu.sync_copy(zero_hbm, stage)
    pltpu.sync_copy(stage, acc.at[stripe])
    plsc.subcore_barrier()

    def run(base, nch):
      pltpu.sync_copy(src_hbm.at[pl.ds(base, nch)], sidx.at[pl.ds(0, nch)])
      pltpu.sync_copy(dst_hbm.at[pl.ds(base, nch)], didx.at[pl.ds(0, nch)])

      def batch(j, carry):
        pltpu.async_copy(y_sp.at[sidx.at[j]], rows, sem0).wait()
        pltpu.sync_copy(rows, acc.at[didx.at[j]], add=True)
        return carry

      lax.fori_loop(0, nch, batch, 0)

    @pl.when(c == 0)
    def _():
      run(s * CH0, CH0)

    @pl.when(c == 1)
    def _():
      run(NS * CH0 + s * CH1, CH1)

    plsc.subcore_barrier()

    pltpu.sync_copy(acc.at[pl.ds(s * RPS, RPS)], stage)
    pltpu.sync_copy(stage, out_hbm.at[c, pl.ds(s * RPS, RPS)])

  return sc_scatter


_sc_scatter_h = _make_sc_scatter(HIDDEN)
_sc_scatter_c = _make_sc_scatter(N_CLASSES)


@jax.jit
def kernel(x, edge_index, edge_weight, W1, att_src1, bias1, W2, att_src2,
           bias2):
  del edge_weight
  src = edge_index[0].astype(jnp.int32)
  dst = edge_index[1].astype(jnp.int32)
  pad = jnp.full((E_PAD - N_EDGES,), N_NODES, jnp.int32)
  src_p = jnp.concatenate([src, pad]).reshape(T_CHUNKS, K)
  dst_p = jnp.concatenate([dst, pad]).reshape(T_CHUNKS, K)

  x_p = jnp.concatenate(
      [x, jnp.zeros((N_ACC - N_NODES, D_FEAT), jnp.float32)])

  zeros_h = jnp.zeros((RPS, HIDDEN), jnp.float32)
  zeros_c = jnp.zeros((RPS, N_CLASSES), jnp.float32)

  y1 = _tc_call(_dense1_body, (N_ACC, HIDDEN), x_p, W1,
                att_src1.reshape(1, HIDDEN))
  p = _sc_scatter_h(y1, src_p, dst_p, zeros_h)
  y2 = _tc_call(_dense2_body, (N_ACC, N_CLASSES), p, y1,
                bias1.reshape(1, HIDDEN), W2, att_src2.reshape(1, N_CLASSES))
  q = _sc_scatter_c(y2, src_p, dst_p, zeros_c)
  out = _tc_call(_final_body, (N_ACC, N_CLASSES), q, y2,
                 bias2.reshape(1, N_CLASSES))
  return out[:N_NODES]
